# Initial kernel scaffold; baseline (speedup 1.0000x reference)
import jax
import jax.numpy as jnp
from jax.experimental import pallas as pl


def kernel(batch_n_nodes, prob, num_nodes):
    raise NotImplementedError("write your pallas kernel here")


# speedup vs baseline: 2.0747x; 2.0747x over previous
import functools

import jax
import jax.numpy as jnp
from jax import lax
from jax.experimental import pallas as pl
from jax.experimental.pallas import tpu as pltpu
from jax.experimental.pallas import tpu_sc as plsc

EPS = 1e-30
LANES = 16


def _log_vec(v):
    bits = plsc.bitcast(v, jnp.int32)
    e = lax.shift_right_logical(bits, jnp.int32(23)) - jnp.int32(127)
    m = plsc.bitcast(
        lax.bitwise_or(lax.bitwise_and(bits, jnp.int32(0x007FFFFF)), jnp.int32(0x3F800000)),
        jnp.float32,
    )
    ef = e.astype(jnp.float32)
    cond = m > jnp.float32(1.4142135)
    m = jnp.where(cond, m * jnp.float32(0.5), m)
    ef = jnp.where(cond, ef + jnp.float32(1.0), ef)
    s = (m - jnp.float32(1.0)) / (m + jnp.float32(1.0))
    z = s * s
    logm = s * (
        jnp.float32(2.0)
        + z * (jnp.float32(0.6666667) + z * (jnp.float32(0.4) + z * jnp.float32(0.28571429)))
    )
    return ef * jnp.float32(0.69314718) + logm


def _sc_info():
    try:
        info = plsc.get_sparse_core_info()
        return info.num_cores, info.num_subcores
    except Exception:
        return 2, 16


def kernel(batch_n_nodes, prob, num_nodes):
    del num_nodes
    batch = batch_n_nodes.shape[0]
    nbuckets = prob.shape[0]
    num_cores, num_subcores = _sc_info()
    nw = num_cores * num_subcores
    assert batch % (8 * nw) == 0
    chunk = batch // nw
    mesh = plsc.VectorSubcoreMesh(core_axis_name="c", subcore_axis_name="s")

    @functools.partial(
        pl.kernel,
        mesh=mesh,
        out_type=jax.ShapeDtypeStruct((batch,), jnp.float32),
        compiler_params=pltpu.CompilerParams(needs_layout_passes=False),
        scratch_types=[
            pltpu.VMEM((nbuckets,), jnp.float32),
            pltpu.VMEM((chunk,), jnp.int32),
            pltpu.VMEM((chunk,), jnp.float32),
        ],
    )
    def run(idx_hbm, prob_hbm, out_hbm, table_v, idx_v, out_v):
        wid = lax.axis_index("s") * num_cores + lax.axis_index("c")
        base = wid * chunk
        pltpu.sync_copy(prob_hbm, table_v)
        pltpu.sync_copy(idx_hbm.at[pl.ds(base, chunk)], idx_v)
        for i in range(nbuckets // LANES):
            sl = pl.ds(i * LANES, LANES)
            table_v[sl] = _log_vec(table_v[sl] + jnp.float32(EPS))
        for j in range(chunk // LANES):
            sl = pl.ds(j * LANES, LANES)
            out_v[sl] = plsc.load_gather(table_v, [idx_v[sl]])
        pltpu.sync_copy(out_v, out_hbm.at[pl.ds(base, chunk)])

    return run(batch_n_nodes, prob)


# speedup vs baseline: 2.1457x; 1.0342x over previous
import functools

import jax
import jax.numpy as jnp
from jax import lax
from jax.experimental import pallas as pl
from jax.experimental.pallas import tpu as pltpu
from jax.experimental.pallas import tpu_sc as plsc

EPS = 1e-30
LANES = 16


def _log_vec(v):
    bits = plsc.bitcast(v, jnp.int32)
    e = lax.shift_right_logical(bits, jnp.int32(23)) - jnp.int32(127)
    m = plsc.bitcast(
        lax.bitwise_or(lax.bitwise_and(bits, jnp.int32(0x007FFFFF)), jnp.int32(0x3F800000)),
        jnp.float32,
    )
    ef = e.astype(jnp.float32)
    cond = m > jnp.float32(1.4142135)
    m = jnp.where(cond, m * jnp.float32(0.5), m)
    ef = jnp.where(cond, ef + jnp.float32(1.0), ef)
    s = (m - jnp.float32(1.0)) / (m + jnp.float32(1.0))
    z = s * s
    logm = s * (
        jnp.float32(2.0)
        + z * (jnp.float32(0.6666667) + z * (jnp.float32(0.4) + z * jnp.float32(0.28571429)))
    )
    return ef * jnp.float32(0.69314718) + logm


def _sc_info():
    try:
        info = plsc.get_sparse_core_info()
        return info.num_cores, info.num_subcores
    except Exception:
        return 2, 16


def kernel(batch_n_nodes, prob, num_nodes):
    del num_nodes
    batch = batch_n_nodes.shape[0]
    nbuckets = prob.shape[0]
    num_cores, num_subcores = _sc_info()
    nw = num_cores * num_subcores
    assert batch % (8 * nw) == 0
    chunk = batch // nw
    mesh = plsc.VectorSubcoreMesh(core_axis_name="c", subcore_axis_name="s")

    @functools.partial(
        pl.kernel,
        mesh=mesh,
        out_type=jax.ShapeDtypeStruct((batch,), jnp.float32),
        compiler_params=pltpu.CompilerParams(needs_layout_passes=False),
        scratch_types=[
            pltpu.VMEM((nbuckets,), jnp.float32),
            pltpu.VMEM((chunk,), jnp.int32),
            pltpu.VMEM((chunk,), jnp.float32),
            pltpu.SemaphoreType.DMA,
            pltpu.SemaphoreType.DMA,
        ],
    )
    def run(idx_hbm, prob_hbm, out_hbm, table_v, idx_v, out_v, sem_p, sem_i):
        wid = lax.axis_index("s") * num_cores + lax.axis_index("c")
        base = wid * chunk
        cp_p = pltpu.async_copy(prob_hbm, table_v, sem_p)
        cp_i = pltpu.async_copy(idx_hbm.at[pl.ds(base, chunk)], idx_v, sem_i)
        cp_p.wait()
        for i in range(nbuckets // LANES):
            sl = pl.ds(i * LANES, LANES)
            table_v[sl] = _log_vec(table_v[sl] + jnp.float32(EPS))
        cp_i.wait()
        for j in range(chunk // LANES):
            sl = pl.ds(j * LANES, LANES)
            out_v[sl] = plsc.load_gather(table_v, [idx_v[sl]])
        pltpu.sync_copy(out_v, out_hbm.at[pl.ds(base, chunk)])

    return run(batch_n_nodes, prob)


# speedup vs baseline: 2.2853x; 1.0651x over previous
import functools

import jax
import jax.numpy as jnp
from jax import lax
from jax.experimental import pallas as pl
from jax.experimental.pallas import tpu as pltpu
from jax.experimental.pallas import tpu_sc as plsc

EPS = 1e-30
LANES = 16


def _log_vec(v):
    bits = plsc.bitcast(v, jnp.int32)
    e = lax.shift_right_logical(bits, jnp.int32(23)) - jnp.int32(127)
    m = plsc.bitcast(
        lax.bitwise_or(lax.bitwise_and(bits, jnp.int32(0x007FFFFF)), jnp.int32(0x3F800000)),
        jnp.float32,
    )
    ef = e.astype(jnp.float32)
    cond = m > jnp.float32(1.4142135)
    m = jnp.where(cond, m * jnp.float32(0.5), m)
    ef = jnp.where(cond, ef + jnp.float32(1.0), ef)
    s = (m - jnp.float32(1.0)) / (m + jnp.float32(1.0))
    z = s * s
    logm = s * (
        jnp.float32(2.0)
        + z * (jnp.float32(0.6666667) + z * (jnp.float32(0.4) + z * jnp.float32(0.28571429)))
    )
    return ef * jnp.float32(0.69314718) + logm


def _sc_info():
    try:
        info = plsc.get_sparse_core_info()
        return info.num_cores, info.num_subcores
    except Exception:
        return 2, 16


def kernel(batch_n_nodes, prob, num_nodes):
    del num_nodes
    batch = batch_n_nodes.shape[0]
    nbuckets = prob.shape[0]
    num_cores, num_subcores = 1, _sc_info()[1]
    nw = num_cores * num_subcores
    assert batch % (8 * nw) == 0
    chunk = batch // nw
    mesh = plsc.VectorSubcoreMesh(core_axis_name="c", subcore_axis_name="s", num_cores=1)

    @functools.partial(
        pl.kernel,
        mesh=mesh,
        out_type=jax.ShapeDtypeStruct((batch,), jnp.float32),
        compiler_params=pltpu.CompilerParams(needs_layout_passes=False),
        scratch_types=[
            pltpu.VMEM((nbuckets,), jnp.float32),
            pltpu.VMEM((chunk,), jnp.int32),
            pltpu.VMEM((chunk,), jnp.float32),
            pltpu.SemaphoreType.DMA,
            pltpu.SemaphoreType.DMA,
        ],
    )
    def run(idx_hbm, prob_hbm, out_hbm, table_v, idx_v, out_v, sem_p, sem_i):
        wid = lax.axis_index("s") * num_cores + lax.axis_index("c")
        base = wid * chunk
        cp_p = pltpu.async_copy(prob_hbm, table_v, sem_p)
        cp_i = pltpu.async_copy(idx_hbm.at[pl.ds(base, chunk)], idx_v, sem_i)
        cp_p.wait()
        for i in range(nbuckets // LANES):
            sl = pl.ds(i * LANES, LANES)
            table_v[sl] = _log_vec(table_v[sl] + jnp.float32(EPS))
        cp_i.wait()
        for j in range(chunk // LANES):
            sl = pl.ds(j * LANES, LANES)
            out_v[sl] = plsc.load_gather(table_v, [idx_v[sl]])
        pltpu.sync_copy(out_v, out_hbm.at[pl.ds(base, chunk)])

    return run(batch_n_nodes, prob)


# speedup vs baseline: 2.2962x; 1.0047x over previous
import functools

import jax
import jax.numpy as jnp
from jax import lax
from jax.experimental import pallas as pl
from jax.experimental.pallas import tpu as pltpu
from jax.experimental.pallas import tpu_sc as plsc

EPS = 1e-30
LANES = 16


def _log_vec(v):
    bits = plsc.bitcast(v, jnp.int32)
    e = lax.shift_right_logical(bits, jnp.int32(23)) - jnp.int32(127)
    m = plsc.bitcast(
        lax.bitwise_or(lax.bitwise_and(bits, jnp.int32(0x007FFFFF)), jnp.int32(0x3F800000)),
        jnp.float32,
    )
    ef = e.astype(jnp.float32)
    cond = m > jnp.float32(1.4142135)
    m = jnp.where(cond, m * jnp.float32(0.5), m)
    ef = jnp.where(cond, ef + jnp.float32(1.0), ef)
    s = (m - jnp.float32(1.0)) / (m + jnp.float32(1.0))
    z = s * s
    logm = s * (
        jnp.float32(2.0)
        + z * (jnp.float32(0.6666667) + z * (jnp.float32(0.4) + z * jnp.float32(0.28571429)))
    )
    return ef * jnp.float32(0.69314718) + logm


def _sc_info():
    try:
        info = plsc.get_sparse_core_info()
        return info.num_cores, info.num_subcores
    except Exception:
        return 2, 16


def kernel(batch_n_nodes, prob, num_nodes):
    del num_nodes
    batch = batch_n_nodes.shape[0]
    nbuckets = prob.shape[0]
    num_cores, num_subcores = 1, _sc_info()[1]
    nw = num_cores * num_subcores
    assert batch % (8 * nw) == 0
    chunk = batch // nw
    mesh = plsc.VectorSubcoreMesh(core_axis_name="c", subcore_axis_name="s", num_cores=1)

    @functools.partial(
        pl.kernel,
        mesh=mesh,
        out_type=jax.ShapeDtypeStruct((batch,), jnp.float32),
        compiler_params=pltpu.CompilerParams(
            needs_layout_passes=False, skip_device_barrier=True
        ),
        scratch_types=[
            pltpu.VMEM((nbuckets,), jnp.float32),
            pltpu.VMEM((chunk,), jnp.int32),
            pltpu.VMEM((chunk,), jnp.float32),
            pltpu.SemaphoreType.DMA,
            pltpu.SemaphoreType.DMA,
        ],
    )
    def run(idx_hbm, prob_hbm, out_hbm, table_v, idx_v, out_v, sem_p, sem_i):
        wid = lax.axis_index("s") * num_cores + lax.axis_index("c")
        base = wid * chunk
        cp_p = pltpu.async_copy(prob_hbm, table_v, sem_p)
        cp_i = pltpu.async_copy(idx_hbm.at[pl.ds(base, chunk)], idx_v, sem_i)
        cp_p.wait()
        for i in range(nbuckets // LANES):
            sl = pl.ds(i * LANES, LANES)
            table_v[sl] = _log_vec(table_v[sl] + jnp.float32(EPS))
        cp_i.wait()
        for j in range(chunk // LANES):
            sl = pl.ds(j * LANES, LANES)
            out_v[sl] = plsc.load_gather(table_v, [idx_v[sl]])
        pltpu.sync_copy(out_v, out_hbm.at[pl.ds(base, chunk)])

    return run(batch_n_nodes, prob)


# speedup vs baseline: 2.2982x; 1.0009x over previous
import functools

import jax
import jax.numpy as jnp
from jax import lax
from jax.experimental import pallas as pl
from jax.experimental.pallas import tpu as pltpu
from jax.experimental.pallas import tpu_sc as plsc

EPS = 1e-30
LANES = 16


def _log_vec(v):
    bits = plsc.bitcast(v, jnp.int32)
    e = lax.shift_right_logical(bits, jnp.int32(23)) - jnp.int32(127)
    m = plsc.bitcast(
        lax.bitwise_or(lax.bitwise_and(bits, jnp.int32(0x007FFFFF)), jnp.int32(0x3F800000)),
        jnp.float32,
    )
    ef = e.astype(jnp.float32)
    cond = m > jnp.float32(1.4142135)
    m = jnp.where(cond, m * jnp.float32(0.5), m)
    ef = jnp.where(cond, ef + jnp.float32(1.0), ef)
    s = (m - jnp.float32(1.0)) / (m + jnp.float32(1.0))
    z = s * s
    logm = s * (
        jnp.float32(2.0)
        + z * (jnp.float32(0.6666667) + z * (jnp.float32(0.4) + z * jnp.float32(0.28571429)))
    )
    return ef * jnp.float32(0.69314718) + logm


def _sc_info():
    try:
        info = plsc.get_sparse_core_info()
        return info.num_cores, info.num_subcores
    except Exception:
        return 2, 16


def kernel(batch_n_nodes, prob, num_nodes):
    del num_nodes
    batch = batch_n_nodes.shape[0]
    nbuckets = prob.shape[0]
    num_cores, num_subcores = 1, _sc_info()[1]
    nw = num_cores * num_subcores
    assert batch % (8 * nw) == 0
    chunk = batch // nw
    mesh = plsc.VectorSubcoreMesh(core_axis_name="c", subcore_axis_name="s", num_cores=1)

    @functools.partial(
        pl.kernel,
        mesh=mesh,
        out_type=jax.ShapeDtypeStruct((batch,), jnp.float32),
        compiler_params=pltpu.CompilerParams(
            needs_layout_passes=False, skip_device_barrier=True
        ),
        scratch_types=[
            pltpu.VMEM((nbuckets,), jnp.float32),
            pltpu.VMEM((chunk,), jnp.int32),
            pltpu.VMEM((chunk,), jnp.float32),
            pltpu.SemaphoreType.DMA,
            pltpu.SemaphoreType.DMA,
        ],
    )
    def run(idx_hbm, prob_hbm, out_hbm, table_v, idx_v, out_v, sem_p, sem_i):
        wid = lax.axis_index("s") * num_cores + lax.axis_index("c")
        base = wid * chunk
        cp_p = pltpu.async_copy(prob_hbm, table_v, sem_p)
        cp_i = pltpu.async_copy(idx_hbm.at[pl.ds(base, chunk)], idx_v, sem_i)
        cp_p.wait()
        for i in range(nbuckets // LANES):
            sl = pl.ds(i * LANES, LANES)
            table_v[sl] = _log_vec(table_v[sl] + jnp.float32(EPS))
        cp_i.wait()

        def gather_step(j, carry):
            sl = pl.ds(j * LANES, LANES)
            out_v[sl] = plsc.load_gather(table_v, [idx_v[sl]])
            return carry

        lax.fori_loop(0, chunk // LANES, gather_step, 0, unroll=4)
        pltpu.sync_copy(out_v, out_hbm.at[pl.ds(base, chunk)])

    return run(batch_n_nodes, prob)
